# baseline (speedup 1.0000x reference)
import dataclasses
import functools

import jax
import jax.numpy as jnp
from jax import lax
from jax.experimental import pallas as pl
from jax.experimental.pallas import tpu as pltpu
from jax.experimental.pallas import tpu_sc as plsc

_IGNORE_LABEL = -1
_THRESH = 0.7
_MIN_KEPT = 100000

_NC = 2
_NS = 16
_L = 16
_NW = _NC * _NS

_MIN_I32 = -2147483648

_BITS = (11, 11, 10)
_SHIFTS = (21, 10, 0)


def _sc_compiler_params():
    cp = pltpu.CompilerParams()
    if "needs_layout_passes" in pltpu.CompilerParams.__dataclass_fields__:
        cp = dataclasses.replace(cp, needs_layout_passes=False)
    if "use_tc_tiling_on_sc" in pltpu.CompilerParams.__dataclass_fields__:
        cp = dataclasses.replace(cp, use_tc_tiling_on_sc=True)
    return cp


def _sort_key(x):
    u = lax.bitcast_convert_type(x, jnp.int32)
    sign = lax.shift_right_arithmetic(u, jnp.full(u.shape, 31, jnp.int32))
    return u ^ (sign | _MIN_I32)


def _shr_logical(x, amount):
    return lax.shift_right_logical(x, jnp.full(x.shape, amount, jnp.int32))


@functools.cache
def _stage_pass(n_chunk, stage):
    nb = 1 << _BITS[stage]
    rows = nb // 128
    dshift = _SHIFTS[stage]
    pshift = _SHIFTS[stage - 1] if stage > 0 else None
    qrows = n_chunk // 512 // 4
    mesh = plsc.VectorSubcoreMesh(core_axis_name="c", subcore_axis_name="s")
    scratch = [
        pltpu.VMEM((2, qrows, 512), jnp.float32),
        pltpu.VMEM((2, qrows, 512), jnp.int32),
        pltpu.VMEM((_L, nb + 1), jnp.float32),
        pltpu.VMEM((_L, nb + 1), jnp.float32),
        pltpu.VMEM((rows, 128), jnp.float32),
        pltpu.VMEM((rows, 128), jnp.float32),
    ]
    if stage == 0:
        scratch.append(pltpu.VMEM((2, _L), jnp.float32))
    else:
        scratch.append(pltpu.VMEM((128,), jnp.int32))
    scratch.extend([pltpu.SemaphoreType.DMA] * 6)

    def body(*refs):
        if stage == 0:
            (pred_hbm, tgt_hbm, cnt_out, loss_out, extra_out,
             data_v, tgt_v, cnt_v, loss_v, cnt_r, loss_r, acc_v,
             *sems) = refs
        else:
            (pred_hbm, tgt_hbm, pfx_hbm, cnt_out, loss_out,
             data_v, tgt_v, cnt_v, loss_v, cnt_r, loss_r, pfx_v,
             *sems) = refs
        wid = lax.axis_index("c") * _NS + lax.axis_index("s")
        if stage != 0:
            pfx_dma = pltpu.async_copy(pfx_hbm, pfx_v, sems[4])
        lanes = lax.iota(jnp.int32, _L)
        ones = jnp.ones((_L,), jnp.float32)
        zero = jnp.zeros((_L,), jnp.float32)
        thresh = jnp.full((_L,), _THRESH, jnp.float32)

        def start_quarter(q):
            base = wid * 4 * qrows + q * qrows
            b = q % 2
            return (
                pltpu.async_copy(pred_hbm.at[pl.ds(base, qrows)],
                                 data_v.at[b], sems[b]),
                pltpu.async_copy(tgt_hbm.at[pl.ds(base, qrows)],
                                 tgt_v.at[b], sems[2 + b]),
            )

        inflight = start_quarter(0)

        @pl.loop(0, nb, step=_L)
        def _zero_cols(c):
            for r in range(_L):
                cnt_v[r, pl.ds(c, _L)] = zero
                loss_v[r, pl.ds(c, _L)] = zero

        if stage != 0:
            pfx_dma.wait()
            pfx = pfx_v[pl.ds(0, _L)]
            pfx_scaled = pfx * nb

        acc07 = (zero, zero)
        for q in range(4):
            b = q % 2
            for w in inflight:
                w.wait()
            if q + 1 < 4:
                inflight = start_quarter(q + 1)

            def row_step(rr, row_carry, b=b):
                def step(i, carry=None):
                    if stage == 0:
                        cnt07, loss07 = carry
                    sl = pl.ds(i, _L)
                    x = data_v[b, rr, sl]
                    t = tgt_v[b, rr, sl]
                    key = _sort_key(x)
                    valid = t != jnp.int32(_IGNORE_LABEL)
                    nll = x - x
                    loss = jnp.where(valid, nll, 0.0)
                    if stage == 0:
                        digit = _shr_logical(key, dshift)
                        plsc.addupdate_scatter(cnt_v, [lanes, digit], ones)
                        plsc.addupdate_scatter(loss_v, [lanes, digit], loss)
                        lt = x < thresh
                        cnt07 = cnt07 + jnp.where(lt, 1.0, 0.0)
                        loss07 = loss07 + jnp.where(lt, loss, 0.0)
                        return cnt07, loss07
                    digit = _shr_logical(key, dshift) - pfx_scaled
                    m = lax.lt(lax.bitcast_convert_type(digit, jnp.uint32),
                               jnp.full((_L,), nb, jnp.uint32))
                    plsc.addupdate_scatter(cnt_v, [lanes, digit], ones,
                                           mask=m)
                    plsc.addupdate_scatter(loss_v, [lanes, digit], loss,
                                           mask=m)
                    return carry

                if stage == 0:
                    return plsc.parallel_loop(
                        0, 512, _L, unroll=16, carry=row_carry)(step)
                plsc.parallel_loop(0, 512, _L, unroll=16)(step)
                return row_carry

            if stage == 0:
                acc07 = lax.fori_loop(0, qrows, row_step, acc07)
            else:
                @pl.loop(0, qrows)
                def _rows(rr):
                    row_step(rr, None)

        @pl.loop(0, rows)
        def _red_row(ri):
            @pl.loop(0, 128, step=_L)
            def _red_col(cj):
                c = ri * 128 + cj
                s = cnt_v[0, pl.ds(c, _L)]
                u = loss_v[0, pl.ds(c, _L)]
                for rr in range(1, _L):
                    s = s + cnt_v[rr, pl.ds(c, _L)]
                    u = u + loss_v[rr, pl.ds(c, _L)]
                cnt_r[ri, pl.ds(cj, _L)] = s
                loss_r[ri, pl.ds(cj, _L)] = u

        pltpu.sync_copy(cnt_r, cnt_out.at[wid])
        pltpu.sync_copy(loss_r, loss_out.at[wid])
        if stage == 0:
            acc_v[0] = acc07[0]
            acc_v[1] = acc07[1]
            pltpu.sync_copy(acc_v, extra_out.at[wid])

    out_type = [
        jax.ShapeDtypeStruct((_NW, rows, 128), jnp.float32),
        jax.ShapeDtypeStruct((_NW, rows, 128), jnp.float32),
    ]
    if stage == 0:
        out_type.append(jax.ShapeDtypeStruct((_NW, 2, _L), jnp.float32))
    return pl.kernel(
        body,
        out_type=out_type,
        mesh=mesh,
        scratch_types=scratch,
        compiler_params=_sc_compiler_params(),
    )


@functools.cache
def _control(stage, rank):
    nb = 1 << _BITS[stage]
    rows = nb // 128
    final = stage == len(_BITS) - 1

    def _cum_inclusive(h):
        col_a = lax.broadcasted_iota(jnp.int32, (128, 128), 0)
        col_b = lax.broadcasted_iota(jnp.int32, (128, 128), 1)
        tri = (col_a <= col_b).astype(jnp.float32)
        within = jax.lax.dot_general(
            h, tri, (((1,), (0,)), ((), ())),
            precision=lax.Precision.HIGHEST,
            preferred_element_type=jnp.float32)
        row_tot = jnp.sum(h, axis=1, keepdims=True)
        row_a = lax.broadcasted_iota(jnp.int32, (rows, rows), 0)
        row_b = lax.broadcasted_iota(jnp.int32, (rows, rows), 1)
        strict = (row_b < row_a).astype(jnp.float32)
        prefix = jax.lax.dot_general(
            strict, row_tot, (((1,), (0,)), ((), ())),
            precision=lax.Precision.HIGHEST,
            preferred_element_type=jnp.float32)
        return within + prefix

    def body(*refs):
        if final:
            (cnt_ref, loss_ref, extra_ref, r_ref, pfx_ref, lsum_ref,
             out_ref) = refs
        else:
            (cnt_ref, loss_ref, r_ref, pfx_ref, lsum_ref,
             pfxv_ref, pfxs_ref, rout_ref, lout_ref) = refs
        h = jnp.sum(cnt_ref[...], axis=0)
        lh = jnp.sum(loss_ref[...], axis=0)
        cum = _cum_inclusive(h)
        r = r_ref[0, 0]
        digit = jnp.sum((cum < r.astype(jnp.float32)).astype(jnp.int32))
        fa = lax.broadcasted_iota(jnp.int32, (rows, 128), 0)
        fb = lax.broadcasted_iota(jnp.int32, (rows, 128), 1)
        flat = fa * 128 + fb
        sel = flat < digit
        below = jnp.sum(jnp.where(sel, h, 0.0))
        lsum = lsum_ref[0, 0] + jnp.sum(jnp.where(sel, lh, 0.0))
        pfx_new = pfx_ref[0, 0] * nb + digit
        r_new = r - below.astype(jnp.int32)
        if not final:
            pfxs_ref[0, 0] = pfx_new
            rout_ref[0, 0] = r_new
            lout_ref[0, 0] = lsum
            pfxv_ref[...] = jnp.full((128,), pfx_new, jnp.int32)
        else:
            key = pfx_new
            inv = jnp.bitwise_not(key)
            m = lax.shift_right_arithmetic(inv, jnp.int32(31)) | _MIN_I32
            min_value = lax.bitcast_convert_type(key ^ m, jnp.float32)
            use_stat = min_value > jnp.float32(_THRESH)
            ex = extra_ref[...]
            cnt07 = jnp.sum(ex[:, 0, :])
            loss07 = jnp.sum(ex[:, 1, :])
            cnt_kth = jnp.float32(rank) - r_new.astype(jnp.float32)
            cnt = jnp.where(use_stat, cnt_kth, cnt07)
            loss = jnp.where(use_stat, lsum, loss07)
            out_ref[0, 0] = loss / jnp.maximum(cnt, 1.0)

    smem = pl.BlockSpec(memory_space=pltpu.SMEM)
    vmem = pl.BlockSpec(memory_space=pltpu.VMEM)
    if final:
        return pl.pallas_call(
            body,
            out_shape=jax.ShapeDtypeStruct((1, 1), jnp.float32),
            in_specs=[vmem, vmem, vmem, smem, smem, smem],
            out_specs=smem,
        )
    return pl.pallas_call(
        body,
        out_shape=[
            jax.ShapeDtypeStruct((128,), jnp.int32),
            jax.ShapeDtypeStruct((1, 1), jnp.int32),
            jax.ShapeDtypeStruct((1, 1), jnp.int32),
            jax.ShapeDtypeStruct((1, 1), jnp.float32),
        ],
        in_specs=[vmem, vmem, smem, smem, smem],
        out_specs=[vmem, smem, smem, smem],
    )


def kernel(score, target):
    n = score.size
    pred = score.reshape(n // 512, 512)
    tgt = target.astype(jnp.int32).reshape(n // 512, 512)
    n_chunk = n // _NW
    rank = min(_MIN_KEPT, n - 1) + 1

    r = jnp.full((1, 1), rank, jnp.int32)
    pfx_s = jnp.zeros((1, 1), jnp.int32)
    lsum = jnp.zeros((1, 1), jnp.float32)

    ch, lh, extra = _stage_pass(n_chunk, 0)(pred, tgt)
    pfx_v, pfx_s, r, lsum = _control(0, rank)(ch, lh, r, pfx_s, lsum)
    ch, lh = _stage_pass(n_chunk, 1)(pred, tgt, pfx_v)
    pfx_v, pfx_s, r, lsum = _control(1, rank)(ch, lh, r, pfx_s, lsum)
    ch, lh = _stage_pass(n_chunk, 2)(pred, tgt, pfx_v)
    out = _control(2, rank)(ch, lh, extra, r, pfx_s, lsum)
    return out.reshape(())


# speedup vs baseline: 1.0207x; 1.0207x over previous
import dataclasses
import functools

import jax
import jax.numpy as jnp
from jax import lax
from jax.experimental import pallas as pl
from jax.experimental.pallas import tpu as pltpu
from jax.experimental.pallas import tpu_sc as plsc

_IGNORE_LABEL = -1
_THRESH = 0.7
_MIN_KEPT = 100000

_NC = 2
_NS = 16
_L = 16
_NW = _NC * _NS

_MIN_I32 = -2147483648

_BITS = (11, 11, 10)
_SHIFTS = (21, 10, 0)


def _sc_compiler_params():
    cp = pltpu.CompilerParams()
    if "needs_layout_passes" in pltpu.CompilerParams.__dataclass_fields__:
        cp = dataclasses.replace(cp, needs_layout_passes=False)
    if "use_tc_tiling_on_sc" in pltpu.CompilerParams.__dataclass_fields__:
        cp = dataclasses.replace(cp, use_tc_tiling_on_sc=True)
    return cp


def _sort_key(x):
    u = lax.bitcast_convert_type(x, jnp.int32)
    sign = lax.shift_right_arithmetic(u, jnp.full(u.shape, 31, jnp.int32))
    return u ^ (sign | _MIN_I32)


def _shr_logical(x, amount):
    return lax.shift_right_logical(x, jnp.full(x.shape, amount, jnp.int32))


@functools.cache
def _stage_pass(n_chunk, stage):
    nb = 1 << _BITS[stage]
    rows = nb // 128
    dshift = _SHIFTS[stage]
    pshift = _SHIFTS[stage - 1] if stage > 0 else None
    qrows = n_chunk // 512 // 4
    mesh = plsc.VectorSubcoreMesh(core_axis_name="c", subcore_axis_name="s")
    scratch = [
        pltpu.VMEM((2, qrows, 512), jnp.float32),
        pltpu.VMEM((2, qrows, 512), jnp.int32),
        pltpu.VMEM((_L, nb + 1), jnp.float32),
        pltpu.VMEM((_L, nb + 1), jnp.float32),
        pltpu.VMEM((rows, 128), jnp.float32),
        pltpu.VMEM((rows, 128), jnp.float32),
    ]
    if stage == 0:
        scratch.append(pltpu.VMEM((2, _L), jnp.float32))
    else:
        scratch.append(pltpu.VMEM((128,), jnp.int32))
    scratch.extend([pltpu.SemaphoreType.DMA] * 6)

    def body(*refs):
        if stage == 0:
            (pred_hbm, tgt_hbm, cnt_out, loss_out, extra_out,
             data_v, tgt_v, cnt_v, loss_v, cnt_r, loss_r, acc_v,
             *sems) = refs
        else:
            (pred_hbm, tgt_hbm, pfx_hbm, cnt_out, loss_out,
             data_v, tgt_v, cnt_v, loss_v, cnt_r, loss_r, pfx_v,
             *sems) = refs
        wid = lax.axis_index("c") * _NS + lax.axis_index("s")
        if stage != 0:
            pfx_dma = pltpu.async_copy(pfx_hbm, pfx_v, sems[4])
        lanes = lax.iota(jnp.int32, _L)
        ones = jnp.ones((_L,), jnp.float32)
        zero = jnp.zeros((_L,), jnp.float32)
        thresh = jnp.full((_L,), _THRESH, jnp.float32)

        def start_quarter(q):
            base = wid * 4 * qrows + q * qrows
            b = q % 2
            return (
                pltpu.async_copy(pred_hbm.at[pl.ds(base, qrows)],
                                 data_v.at[b], sems[b]),
                pltpu.async_copy(tgt_hbm.at[pl.ds(base, qrows)],
                                 tgt_v.at[b], sems[2 + b]),
            )

        inflight = start_quarter(0)

        @pl.loop(0, nb, step=_L)
        def _zero_cols(c):
            for r in range(_L):
                cnt_v[r, pl.ds(c, _L)] = zero
                loss_v[r, pl.ds(c, _L)] = zero

        if stage != 0:
            pfx_dma.wait()
            pfx = pfx_v[pl.ds(0, _L)]
            pfx_scaled = pfx * nb

        acc07 = (zero, zero)
        for q in range(4):
            b = q % 2
            for w in inflight:
                w.wait()
            if q + 1 < 4:
                inflight = start_quarter(q + 1)

            def row_step(rr, row_carry, b=b):
                def step(i, carry=None):
                    if stage == 0:
                        cnt07, loss07 = carry
                    sl = pl.ds(i, _L)
                    x = data_v[b, rr, sl]
                    t = tgt_v[b, rr, sl]
                    key = _sort_key(x)
                    valid = t != jnp.int32(_IGNORE_LABEL)
                    nll = x - x
                    loss = jnp.where(valid, nll, 0.0)
                    if stage == 0:
                        digit = _shr_logical(key, dshift)
                        plsc.addupdate_scatter(cnt_v, [lanes, digit], ones)
                        plsc.addupdate_scatter(loss_v, [lanes, digit], loss)
                        lt = x < thresh
                        cnt07 = cnt07 + jnp.where(lt, 1.0, 0.0)
                        loss07 = loss07 + jnp.where(lt, loss, 0.0)
                        return cnt07, loss07
                    digit = _shr_logical(key, dshift) - pfx_scaled
                    m = lax.lt(lax.bitcast_convert_type(digit, jnp.uint32),
                               jnp.full((_L,), nb, jnp.uint32))
                    plsc.addupdate_scatter(cnt_v, [lanes, digit], ones,
                                           mask=m)
                    plsc.addupdate_scatter(loss_v, [lanes, digit], loss,
                                           mask=m)
                    return carry

                if stage == 0:
                    return plsc.parallel_loop(
                        0, 512, _L, unroll=8, carry=row_carry)(step)
                plsc.parallel_loop(0, 512, _L, unroll=8)(step)
                return row_carry

            if stage == 0:
                acc07 = lax.fori_loop(0, qrows, row_step, acc07)
            else:
                @pl.loop(0, qrows)
                def _rows(rr):
                    row_step(rr, None)

        @pl.loop(0, rows)
        def _red_row(ri):
            @pl.loop(0, 128, step=_L)
            def _red_col(cj):
                c = ri * 128 + cj
                s = cnt_v[0, pl.ds(c, _L)]
                u = loss_v[0, pl.ds(c, _L)]
                for rr in range(1, _L):
                    s = s + cnt_v[rr, pl.ds(c, _L)]
                    u = u + loss_v[rr, pl.ds(c, _L)]
                cnt_r[ri, pl.ds(cj, _L)] = s
                loss_r[ri, pl.ds(cj, _L)] = u

        pltpu.sync_copy(cnt_r, cnt_out.at[wid])
        pltpu.sync_copy(loss_r, loss_out.at[wid])
        if stage == 0:
            acc_v[0] = acc07[0]
            acc_v[1] = acc07[1]
            pltpu.sync_copy(acc_v, extra_out.at[wid])

    out_type = [
        jax.ShapeDtypeStruct((_NW, rows, 128), jnp.float32),
        jax.ShapeDtypeStruct((_NW, rows, 128), jnp.float32),
    ]
    if stage == 0:
        out_type.append(jax.ShapeDtypeStruct((_NW, 2, _L), jnp.float32))
    return pl.kernel(
        body,
        out_type=out_type,
        mesh=mesh,
        scratch_types=scratch,
        compiler_params=_sc_compiler_params(),
    )


@functools.cache
def _control(stage, rank):
    nb = 1 << _BITS[stage]
    rows = nb // 128
    final = stage == len(_BITS) - 1

    def _cum_inclusive(h):
        col_a = lax.broadcasted_iota(jnp.int32, (128, 128), 0)
        col_b = lax.broadcasted_iota(jnp.int32, (128, 128), 1)
        tri = (col_a <= col_b).astype(jnp.float32)
        within = jax.lax.dot_general(
            h, tri, (((1,), (0,)), ((), ())),
            precision=lax.Precision.HIGHEST,
            preferred_element_type=jnp.float32)
        row_tot = jnp.sum(h, axis=1, keepdims=True)
        row_a = lax.broadcasted_iota(jnp.int32, (rows, rows), 0)
        row_b = lax.broadcasted_iota(jnp.int32, (rows, rows), 1)
        strict = (row_b < row_a).astype(jnp.float32)
        prefix = jax.lax.dot_general(
            strict, row_tot, (((1,), (0,)), ((), ())),
            precision=lax.Precision.HIGHEST,
            preferred_element_type=jnp.float32)
        return within + prefix

    def body(*refs):
        if final:
            (cnt_ref, loss_ref, extra_ref, r_ref, pfx_ref, lsum_ref,
             out_ref) = refs
        else:
            (cnt_ref, loss_ref, r_ref, pfx_ref, lsum_ref,
             pfxv_ref, pfxs_ref, rout_ref, lout_ref) = refs
        h = jnp.sum(cnt_ref[...], axis=0)
        lh = jnp.sum(loss_ref[...], axis=0)
        cum = _cum_inclusive(h)
        r = r_ref[0, 0]
        digit = jnp.sum((cum < r.astype(jnp.float32)).astype(jnp.int32))
        fa = lax.broadcasted_iota(jnp.int32, (rows, 128), 0)
        fb = lax.broadcasted_iota(jnp.int32, (rows, 128), 1)
        flat = fa * 128 + fb
        sel = flat < digit
        below = jnp.sum(jnp.where(sel, h, 0.0))
        lsum = lsum_ref[0, 0] + jnp.sum(jnp.where(sel, lh, 0.0))
        pfx_new = pfx_ref[0, 0] * nb + digit
        r_new = r - below.astype(jnp.int32)
        if not final:
            pfxs_ref[0, 0] = pfx_new
            rout_ref[0, 0] = r_new
            lout_ref[0, 0] = lsum
            pfxv_ref[...] = jnp.full((128,), pfx_new, jnp.int32)
        else:
            key = pfx_new
            inv = jnp.bitwise_not(key)
            m = lax.shift_right_arithmetic(inv, jnp.int32(31)) | _MIN_I32
            min_value = lax.bitcast_convert_type(key ^ m, jnp.float32)
            use_stat = min_value > jnp.float32(_THRESH)
            ex = extra_ref[...]
            cnt07 = jnp.sum(ex[:, 0, :])
            loss07 = jnp.sum(ex[:, 1, :])
            cnt_kth = jnp.float32(rank) - r_new.astype(jnp.float32)
            cnt = jnp.where(use_stat, cnt_kth, cnt07)
            loss = jnp.where(use_stat, lsum, loss07)
            out_ref[0, 0] = loss / jnp.maximum(cnt, 1.0)

    smem = pl.BlockSpec(memory_space=pltpu.SMEM)
    vmem = pl.BlockSpec(memory_space=pltpu.VMEM)
    if final:
        return pl.pallas_call(
            body,
            out_shape=jax.ShapeDtypeStruct((1, 1), jnp.float32),
            in_specs=[vmem, vmem, vmem, smem, smem, smem],
            out_specs=smem,
        )
    return pl.pallas_call(
        body,
        out_shape=[
            jax.ShapeDtypeStruct((128,), jnp.int32),
            jax.ShapeDtypeStruct((1, 1), jnp.int32),
            jax.ShapeDtypeStruct((1, 1), jnp.int32),
            jax.ShapeDtypeStruct((1, 1), jnp.float32),
        ],
        in_specs=[vmem, vmem, smem, smem, smem],
        out_specs=[vmem, smem, smem, smem],
    )


def kernel(score, target):
    n = score.size
    pred = score.reshape(n // 512, 512)
    tgt = target.astype(jnp.int32).reshape(n // 512, 512)
    n_chunk = n // _NW
    rank = min(_MIN_KEPT, n - 1) + 1

    r = jnp.full((1, 1), rank, jnp.int32)
    pfx_s = jnp.zeros((1, 1), jnp.int32)
    lsum = jnp.zeros((1, 1), jnp.float32)

    ch, lh, extra = _stage_pass(n_chunk, 0)(pred, tgt)
    pfx_v, pfx_s, r, lsum = _control(0, rank)(ch, lh, r, pfx_s, lsum)
    ch, lh = _stage_pass(n_chunk, 1)(pred, tgt, pfx_v)
    pfx_v, pfx_s, r, lsum = _control(1, rank)(ch, lh, r, pfx_s, lsum)
    ch, lh = _stage_pass(n_chunk, 2)(pred, tgt, pfx_v)
    out = _control(2, rank)(ch, lh, extra, r, pfx_s, lsum)
    return out.reshape(())
